# baseline (speedup 1.0000x reference)
import functools

import jax
import jax.numpy as jnp
from jax import lax
from jax.experimental import pallas as pl
from jax.experimental.pallas import tpu as pltpu
from jax.experimental.pallas import tpu_sc as plsc

N = 10000
E = 320000
G = 64
F = 128
H4 = 32
NP = 10240
HW = 64
TRASH = N

NC, NS = 2, 16
NW = NC * NS
EP = NW * NP
EPT = EP // NW
K = 128
NB = EPT // K
RPT = NP // NS
NRING = 2

_f32 = jnp.float32

_MESH = plsc.VectorSubcoreMesh(
    core_axis_name="c", subcore_axis_name="s", num_cores=NC, num_subcores=NS)
_SC_PARAMS = pltpu.CompilerParams(
    needs_layout_passes=False, use_tc_tiling_on_sc=False)


def _zero_1d(ref, nwords):
    z = jnp.zeros((16,), _f32)

    @pl.loop(0, nwords // 16, unroll=8)
    def _(i):
        ref[pl.ds(i * 16, 16)] = z


def _zero_2d(ref, rows, cols):
    z = jnp.zeros((16,), _f32)

    @pl.loop(0, rows)
    def _(r):
        for j in range(cols // 16):
            ref[r, pl.ds(j * 16, 16)] = z


def _hist_reduce_write(acc, shared, slab, out, c, s):
    pltpu.sync_copy(acc, shared.at[s])
    plsc.subcore_barrier()
    base = s * RPT
    pltpu.sync_copy(shared.at[:, pl.ds(base, RPT)], slab)

    @pl.loop(0, RPT // 16)
    def _(i):
        v = jnp.zeros((16,), _f32)
        for k in range(NS):
            v = v + slab[k, pl.ds(i * 16, 16)]
        acc[pl.ds(base + i * 16, 16)] = v

    pltpu.sync_copy(acc.at[pl.ds(base, RPT)], out.at[c, pl.ds(base, RPT)])


@functools.partial(
    pl.kernel,
    out_type=jax.ShapeDtypeStruct((NC, NP), _f32),
    mesh=_MESH,
    compiler_params=_SC_PARAMS,
    scratch_types=[
        pltpu.VMEM((EPT,), jnp.int32),
        pltpu.VMEM((NP,), _f32),
        pltpu.VMEM((NS, RPT), _f32),
        pltpu.VMEM_SHARED((NS, NP), _f32),
    ],
)
def _sc_deg(dst_hbm, out, dstv, acc, slab, shared):
    c = lax.axis_index("c")
    s = lax.axis_index("s")
    wid = c * NS + s
    pltpu.sync_copy(dst_hbm.at[wid], dstv)
    _zero_1d(acc, NP)
    one = jnp.ones((16,), _f32)

    @pl.loop(0, EPT // 16, unroll=4)
    def _(i):
        d = dstv[pl.ds(i * 16, 16)]
        plsc.addupdate_scatter(acc, [d], one)

    _hist_reduce_write(acc, shared, slab, out, c, s)


@functools.partial(
    pl.kernel,
    out_type=jax.ShapeDtypeStruct((NC, NP), _f32),
    mesh=_MESH,
    compiler_params=_SC_PARAMS,
    scratch_types=[
        pltpu.VMEM((EPT,), jnp.int32),
        pltpu.VMEM((EPT,), jnp.int32),
        pltpu.VMEM((NP,), _f32),
        pltpu.VMEM((NP,), _f32),
        pltpu.VMEM((NS, RPT), _f32),
        pltpu.VMEM_SHARED((NS, NP), _f32),
    ],
)
def _sc_hist_gather(val_hbm, src_hbm, dst_hbm, out, srcv, dstv, valv, acc,
                    slab, shared):
    c = lax.axis_index("c")
    s = lax.axis_index("s")
    wid = c * NS + s
    pltpu.sync_copy(src_hbm.at[wid], srcv)
    pltpu.sync_copy(dst_hbm.at[wid], dstv)
    pltpu.sync_copy(val_hbm, valv)
    _zero_1d(acc, NP)

    @pl.loop(0, EPT // 16, unroll=4)
    def _(i):
        sv = srcv[pl.ds(i * 16, 16)]
        d = dstv[pl.ds(i * 16, 16)]
        v = plsc.load_gather(valv, [sv])
        plsc.addupdate_scatter(acc, [d], v)

    _hist_reduce_write(acc, shared, slab, out, c, s)


@functools.partial(
    pl.kernel,
    out_type=(
        jax.ShapeDtypeStruct((NC, NP), _f32),
        jax.ShapeDtypeStruct((NW, EPT), jnp.int32),
        jax.ShapeDtypeStruct((NW, EPT), jnp.int32),
    ),
    mesh=_MESH,
    compiler_params=_SC_PARAMS,
    scratch_types=[
        pltpu.VMEM((EPT,), jnp.int32),
        pltpu.VMEM((EPT,), jnp.int32),
        pltpu.VMEM((EPT,), jnp.int32),
        pltpu.VMEM((EPT,), jnp.int32),
        pltpu.VMEM((NP,), _f32),
        pltpu.VMEM((NP,), _f32),
        pltpu.VMEM((NS, RPT), _f32),
        pltpu.VMEM_SHARED((NS, NP), _f32),
    ],
)
def _sc_prune(src_hbm, dst_hbm, mask_hbm, out_deg, out_src, out_dst,
              srcv, dstv, spv, dpv, maskv, acc, slab, shared):
    c = lax.axis_index("c")
    s = lax.axis_index("s")
    wid = c * NS + s
    pltpu.sync_copy(src_hbm.at[wid], srcv)
    pltpu.sync_copy(dst_hbm.at[wid], dstv)
    pltpu.sync_copy(mask_hbm, maskv)
    _zero_1d(acc, NP)
    one = jnp.ones((16,), _f32)
    lane = lax.iota(jnp.int32, 16)
    zero_i = jnp.zeros((16,), jnp.int32)

    @pl.loop(0, EPT // 16, unroll=4)
    def _(i):
        sv = srcv[pl.ds(i * 16, 16)]
        d = dstv[pl.ds(i * 16, 16)]
        ms = plsc.load_gather(maskv, [sv])
        md = plsc.load_gather(maskv, [d])
        em = (ms > 0.5) & (md > 0.5)
        trash = TRASH + lax.rem(i, 15) * 16 + lane
        sp = jnp.where(em, sv, zero_i)
        dp = jnp.where(em, d, trash)
        spv[pl.ds(i * 16, 16)] = sp
        dpv[pl.ds(i * 16, 16)] = dp
        plsc.addupdate_scatter(acc, [dp], one)

    pltpu.sync_copy(spv, out_src.at[wid])
    pltpu.sync_copy(dpv, out_dst.at[wid])
    _hist_reduce_write(acc, shared, slab, out_deg, c, s)


@functools.partial(
    pl.kernel,
    out_type=(
        jax.ShapeDtypeStruct((NC, NP, HW), _f32),
        jax.ShapeDtypeStruct((NC, NP, HW), _f32),
    ),
    mesh=_MESH,
    compiler_params=_SC_PARAMS,
    scratch_types=[
        pltpu.VMEM((NB, K), jnp.int32),
        pltpu.VMEM((NB, K), jnp.int32),
        [pltpu.VMEM((K, HW), _f32)] * NRING,
        [pltpu.SemaphoreType.DMA] * NRING,
        [pltpu.SemaphoreType.DMA] * NRING,
        pltpu.VMEM_SHARED((NP, HW), _f32),
    ],
)
def _sc_conv(t0_hbm, t1_hbm, src_hbm, dst_hbm, outA, outB, sidx, didx,
             bufs, gsems, ssems, shared):
    c = lax.axis_index("c")
    s = lax.axis_index("s")
    wid = c * NS + s
    pltpu.sync_copy(src_hbm.at[wid], sidx)
    pltpu.sync_copy(dst_hbm.at[wid], didx)
    base = s * RPT

    for tbl, outp in ((t0_hbm, outA), (t1_hbm, outB)):
        _zero_2d(bufs[0], K, HW)
        for j in range(RPT // K):
            pltpu.sync_copy(bufs[0], shared.at[pl.ds(base + j * K, K)])
        plsc.subcore_barrier()

        buf0, buf1 = bufs[0], bufs[1]
        sem0, sem1 = gsems[0], gsems[1]
        pltpu.async_copy(tbl.at[sidx.at[0]], buf0, sem0)

        @pl.loop(0, NB // 2 - 1)
        def _(t):
            b = 2 * t
            pltpu.make_async_copy(tbl.at[sidx.at[b]], buf0, sem0).wait()
            pltpu.async_copy(tbl.at[sidx.at[b + 1]], buf1, sem1)
            pltpu.sync_copy(buf0, shared.at[didx.at[b]], add=True)
            pltpu.make_async_copy(tbl.at[sidx.at[b + 1]], buf1, sem1).wait()
            pltpu.async_copy(tbl.at[sidx.at[b + 2]], buf0, sem0)
            pltpu.sync_copy(buf1, shared.at[didx.at[b + 1]], add=True)

        b = NB - 2
        pltpu.make_async_copy(tbl.at[sidx.at[b]], buf0, sem0).wait()
        pltpu.async_copy(tbl.at[sidx.at[b + 1]], buf1, sem1)
        pltpu.sync_copy(buf0, shared.at[didx.at[b]], add=True)
        pltpu.make_async_copy(tbl.at[sidx.at[b + 1]], buf1, sem1).wait()
        pltpu.sync_copy(buf1, shared.at[didx.at[b + 1]], add=True)

        plsc.subcore_barrier()
        pltpu.sync_copy(shared.at[pl.ds(base, RPT)],
                        outp.at[c, pl.ds(base, RPT)])


def _tc_call(body, out_shapes):
    return pl.pallas_call(body, out_shape=out_shapes)


def _gelu(v):
    return 0.5 * v * (1.0 + lax.erf(v * (2.0 ** -0.5)))


def _cat(a0A, a1A, a0B, a1B):
    return jnp.concatenate([a0A[...] + a1A[...], a0B[...] + a1B[...]], axis=1)


def _tc1_body(d0, d1, x, w1, dis_o, hs1_o):
    deg = d0[...] + d1[...] + 1.0
    dis = lax.rsqrt(deg)
    h = jnp.dot(x[...], w1[...], preferred_element_type=_f32)
    dis_o[...] = dis
    hs1_o[...] = h * dis


def _tc2_body(a0A, a1A, a0B, a1B, hs1, dis, b1, w2, x1_o, hs2_o):
    agg = _cat(a0A, a1A, a0B, a1B) + hs1[...]
    x1 = _gelu(dis[...] * agg + b1[...])
    h2 = jnp.dot(x1, w2[...], preferred_element_type=_f32)
    x1_o[...] = x1
    hs2_o[...] = h2 * dis[...]


def _tc3_body(a0A, a1A, a0B, a1B, hs2, dis, b2, x1, lnw, lnb, wp,
              xl_o, hsp_o):
    agg = _cat(a0A, a1A, a0B, a1B) + hs2[...]
    x2 = _gelu(dis[...] * agg + b2[...])
    y = x1[...] + x2
    u = jnp.mean(y, axis=1, keepdims=True)
    v = jnp.mean((y - u) ** 2, axis=1, keepdims=True)
    xl = lnw[...] * ((y - u) / jnp.sqrt(v + 1e-12)) + lnb[...]
    sp = jnp.dot(xl, wp[...], preferred_element_type=_f32)
    xl_o[...] = xl
    hsp_o[...] = sp * dis[...]


def _tc4_body(a0, a1, hsp, dis, bp, batch, xl, xp_o, mask_o):
    spre = dis[...] * (a0[...] + a1[...] + hsp[...]) + bp[0, 0]
    bcol = batch[...]
    gid = lax.broadcasted_iota(jnp.int32, (NP, G), 1)
    onehot = bcol == gid
    valid = bcol < G
    NEG = jnp.float32(-1e30)
    m = jnp.max(jnp.where(onehot, spre, NEG), axis=0, keepdims=True)
    mb = jnp.sum(jnp.where(onehot, m, 0.0), axis=1, keepdims=True)
    ex = jnp.exp(spre - mb)
    den = jnp.sum(jnp.where(onehot, ex, 0.0), axis=0, keepdims=True)
    denb = jnp.sum(jnp.where(onehot, den, 0.0), axis=1, keepdims=True)
    score = jnp.where(valid, ex / (denb + 1e-16), 0.0)
    smax = jnp.max(jnp.where(onehot, score, NEG), axis=0, keepdims=True)
    smaxb = jnp.sum(jnp.where(onehot, smax, 0.0), axis=1, keepdims=True)
    smin = jnp.minimum(smaxb - 1e-7, 0.001)
    mask = (score > smin) & valid
    xp_o[...] = xl[...] * score
    mask_o[...] = mask.astype(_f32)


def _tc5_body(d0, d1, xp, g_o, dis3_o):
    deg3 = d0[...] + d1[...] + 1.0
    dis3 = lax.rsqrt(deg3)
    dis3_o[...] = dis3
    g_o[...] = xp[...] * dis3


def _tc7_body(a0A, a1A, a0B, a1B, g, dis3, w3, b3, mask, batch_row, out_o):
    agg = _cat(a0A, a1A, a0B, a1B) + g[...]
    h3 = jnp.dot(agg, w3[...], preferred_element_type=_f32)
    x3 = _gelu(dis3[...] * h3 + b3[...])
    x3 = jnp.where(mask[...] > 0.5, x3, 0.0)
    gid = lax.broadcasted_iota(jnp.int32, (G, NP), 0)
    onehot_t = (batch_row[...] == gid).astype(_f32)
    out_o[...] = jnp.dot(onehot_t, x3, preferred_element_type=_f32)


def kernel(x, edge_index, batch, W1, b1, W2, b2, ln_w, ln_b, Wp, bp, W3, b3):
    npad = EP - E
    trash_ids = TRASH + (jnp.arange(npad, dtype=jnp.int32) % (NP - N))
    src = jnp.concatenate([edge_index[0], jnp.zeros((npad,), jnp.int32)])
    dst = jnp.concatenate([edge_index[1], trash_ids])
    src2 = src.reshape(NW, EPT)
    dst2 = dst.reshape(NW, EPT)
    src3 = src.reshape(NW, NB, K)
    dst3 = dst.reshape(NW, NB, K)
    xpad = jnp.pad(x, ((0, NP - N), (0, 0)))
    batch_p = jnp.pad(batch, (0, NP - N), constant_values=G)
    batch_col = batch_p.reshape(NP, 1)
    batch_row = batch_p.reshape(1, NP)
    b1r = b1.reshape(1, F)
    b2r = b2.reshape(1, F)
    lnwr = ln_w.reshape(1, F)
    lnbr = ln_b.reshape(1, F)
    bpr = bp.reshape(1, 1)
    b3r = b3.reshape(1, H4)

    deg = _sc_deg(dst2)
    d0 = deg[0].reshape(NP, 1)
    d1 = deg[1].reshape(NP, 1)

    dis, hs1 = _tc_call(_tc1_body, (
        jax.ShapeDtypeStruct((NP, 1), _f32),
        jax.ShapeDtypeStruct((NP, F), _f32)))(d0, d1, xpad, W1)

    aggA, aggB = _sc_conv(hs1[:, :HW], hs1[:, HW:], src3, dst3)
    x1, hs2 = _tc_call(_tc2_body, (
        jax.ShapeDtypeStruct((NP, F), _f32),
        jax.ShapeDtypeStruct((NP, F), _f32)))(
            aggA[0], aggA[1], aggB[0], aggB[1], hs1, dis, b1r, W2)

    aggA, aggB = _sc_conv(hs2[:, :HW], hs2[:, HW:], src3, dst3)
    xl, hsp = _tc_call(_tc3_body, (
        jax.ShapeDtypeStruct((NP, F), _f32),
        jax.ShapeDtypeStruct((NP, 1), _f32)))(
            aggA[0], aggA[1], aggB[0], aggB[1], hs2, dis, b2r, x1,
            lnwr, lnbr, Wp)

    aggp = _sc_hist_gather(hsp.reshape(NP), src2, dst2)
    xp, maskf = _tc_call(_tc4_body, (
        jax.ShapeDtypeStruct((NP, F), _f32),
        jax.ShapeDtypeStruct((NP, 1), _f32)))(
            aggp[0].reshape(NP, 1), aggp[1].reshape(NP, 1), hsp, dis, bpr,
            batch_col, xl)

    deg3, srcp, dstp = _sc_prune(src2, dst2, maskf.reshape(NP))
    g, dis3 = _tc_call(_tc5_body, (
        jax.ShapeDtypeStruct((NP, F), _f32),
        jax.ShapeDtypeStruct((NP, 1), _f32)))(
            deg3[0].reshape(NP, 1), deg3[1].reshape(NP, 1), xp)

    aggA, aggB = _sc_conv(g[:, :HW], g[:, HW:],
                          srcp.reshape(NW, NB, K), dstp.reshape(NW, NB, K))
    out = _tc_call(_tc7_body, jax.ShapeDtypeStruct((G, H4), _f32))(
        aggA[0], aggA[1], aggB[0], aggB[1], g, dis3, W3, b3r, maskf,
        batch_row)
    return out


# speedup vs baseline: 1.6568x; 1.6568x over previous
import functools

import jax
import jax.numpy as jnp
from jax import lax
from jax.experimental import pallas as pl
from jax.experimental.pallas import tpu as pltpu
from jax.experimental.pallas import tpu_sc as plsc

N = 10000
E = 320000
G = 64
F = 128
H4 = 32
NP = 10240
HW = 64
TRASH = N

NC, NS = 2, 16
NW = NC * NS
EP = NW * NP
EPT = EP // NW
K = 128
NB = EPT // K
RPT = NP // NS
NRING = 2

_f32 = jnp.float32

_MESH = plsc.VectorSubcoreMesh(
    core_axis_name="c", subcore_axis_name="s", num_cores=NC, num_subcores=NS)
_SC_PARAMS = pltpu.CompilerParams(
    needs_layout_passes=False, use_tc_tiling_on_sc=False)


def _zero_1d(ref, nwords):
    z = jnp.zeros((16,), _f32)

    @pl.loop(0, nwords // 16, unroll=8)
    def _(i):
        ref[pl.ds(i * 16, 16)] = z


def _zero_2d(ref, rows, cols):
    z = jnp.zeros((16,), _f32)

    @pl.loop(0, rows)
    def _(r):
        for j in range(cols // 16):
            ref[r, pl.ds(j * 16, 16)] = z


def _hist_reduce_write(acc, shared, slab, out, c, s):
    pltpu.sync_copy(acc, shared.at[s])
    plsc.subcore_barrier()
    base = s * RPT
    pltpu.sync_copy(shared.at[:, pl.ds(base, RPT)], slab)

    @pl.loop(0, RPT // 16)
    def _(i):
        v = jnp.zeros((16,), _f32)
        for k in range(NS):
            v = v + slab[k, pl.ds(i * 16, 16)]
        acc[pl.ds(base + i * 16, 16)] = v

    pltpu.sync_copy(acc.at[pl.ds(base, RPT)], out.at[c, pl.ds(base, RPT)])


@functools.partial(
    pl.kernel,
    out_type=jax.ShapeDtypeStruct((NC, NP), _f32),
    mesh=_MESH,
    compiler_params=_SC_PARAMS,
    scratch_types=[
        pltpu.VMEM((EPT,), jnp.int32),
        pltpu.VMEM((NP,), _f32),
        pltpu.VMEM((NS, RPT), _f32),
        pltpu.VMEM_SHARED((NS, NP), _f32),
    ],
)
def _sc_deg(dst_hbm, out, dstv, acc, slab, shared):
    c = lax.axis_index("c")
    s = lax.axis_index("s")
    wid = c * NS + s
    pltpu.sync_copy(dst_hbm.at[wid], dstv)
    _zero_1d(acc, NP)
    one = jnp.ones((16,), _f32)

    @pl.loop(0, EPT // 16, unroll=4)
    def _(i):
        d = dstv[pl.ds(i * 16, 16)]
        plsc.addupdate_scatter(acc, [d], one)

    _hist_reduce_write(acc, shared, slab, out, c, s)


@functools.partial(
    pl.kernel,
    out_type=jax.ShapeDtypeStruct((NC, NP), _f32),
    mesh=_MESH,
    compiler_params=_SC_PARAMS,
    scratch_types=[
        pltpu.VMEM((EPT,), jnp.int32),
        pltpu.VMEM((EPT,), jnp.int32),
        pltpu.VMEM((NP,), _f32),
        pltpu.VMEM((NP,), _f32),
        pltpu.VMEM((NS, RPT), _f32),
        pltpu.VMEM_SHARED((NS, NP), _f32),
    ],
)
def _sc_hist_gather(val_hbm, src_hbm, dst_hbm, out, srcv, dstv, valv, acc,
                    slab, shared):
    c = lax.axis_index("c")
    s = lax.axis_index("s")
    wid = c * NS + s
    pltpu.sync_copy(src_hbm.at[wid], srcv)
    pltpu.sync_copy(dst_hbm.at[wid], dstv)
    pltpu.sync_copy(val_hbm, valv)
    _zero_1d(acc, NP)

    @pl.loop(0, EPT // 16, unroll=4)
    def _(i):
        sv = srcv[pl.ds(i * 16, 16)]
        d = dstv[pl.ds(i * 16, 16)]
        v = plsc.load_gather(valv, [sv])
        plsc.addupdate_scatter(acc, [d], v)

    _hist_reduce_write(acc, shared, slab, out, c, s)


@functools.partial(
    pl.kernel,
    out_type=(
        jax.ShapeDtypeStruct((NC, NP), _f32),
        jax.ShapeDtypeStruct((NW, EPT), jnp.int32),
        jax.ShapeDtypeStruct((NW, EPT), jnp.int32),
    ),
    mesh=_MESH,
    compiler_params=_SC_PARAMS,
    scratch_types=[
        pltpu.VMEM((EPT,), jnp.int32),
        pltpu.VMEM((EPT,), jnp.int32),
        pltpu.VMEM((EPT,), jnp.int32),
        pltpu.VMEM((EPT,), jnp.int32),
        pltpu.VMEM((NP,), _f32),
        pltpu.VMEM((NP,), _f32),
        pltpu.VMEM((NS, RPT), _f32),
        pltpu.VMEM_SHARED((NS, NP), _f32),
    ],
)
def _sc_prune(src_hbm, dst_hbm, mask_hbm, out_deg, out_src, out_dst,
              srcv, dstv, spv, dpv, maskv, acc, slab, shared):
    c = lax.axis_index("c")
    s = lax.axis_index("s")
    wid = c * NS + s
    pltpu.sync_copy(src_hbm.at[wid], srcv)
    pltpu.sync_copy(dst_hbm.at[wid], dstv)
    pltpu.sync_copy(mask_hbm, maskv)
    _zero_1d(acc, NP)
    one = jnp.ones((16,), _f32)
    lane = lax.iota(jnp.int32, 16)
    zero_i = jnp.zeros((16,), jnp.int32)

    @pl.loop(0, EPT // 16, unroll=4)
    def _(i):
        sv = srcv[pl.ds(i * 16, 16)]
        d = dstv[pl.ds(i * 16, 16)]
        ms = plsc.load_gather(maskv, [sv])
        md = plsc.load_gather(maskv, [d])
        em = (ms > 0.5) & (md > 0.5)
        trash = TRASH + lax.rem(i, 15) * 16 + lane
        sp = jnp.where(em, sv, zero_i)
        dp = jnp.where(em, d, trash)
        spv[pl.ds(i * 16, 16)] = sp
        dpv[pl.ds(i * 16, 16)] = dp
        plsc.addupdate_scatter(acc, [dp], one)

    pltpu.sync_copy(spv, out_src.at[wid])
    pltpu.sync_copy(dpv, out_dst.at[wid])
    _hist_reduce_write(acc, shared, slab, out_deg, c, s)


@functools.partial(
    pl.kernel,
    out_type=(
        jax.ShapeDtypeStruct((NC, NP, HW), _f32),
        jax.ShapeDtypeStruct((NC, NP, HW), _f32),
    ),
    mesh=_MESH,
    compiler_params=_SC_PARAMS,
    scratch_types=[
        pltpu.VMEM((NB, K), jnp.int32),
        pltpu.VMEM((NB, K), jnp.int32),
        [pltpu.VMEM((K, HW), _f32)] * NRING,
        [pltpu.SemaphoreType.DMA] * NRING,
        [pltpu.SemaphoreType.DMA] * NRING,
        pltpu.VMEM_SHARED((NP, HW), _f32),
    ],
)
def _sc_conv(t0_hbm, t1_hbm, src_hbm, dst_hbm, outA, outB, sidx, didx,
             bufs, gsems, ssems, shared):
    c = lax.axis_index("c")
    s = lax.axis_index("s")
    wid = c * NS + s
    pltpu.sync_copy(src_hbm.at[wid], sidx)
    pltpu.sync_copy(dst_hbm.at[wid], didx)
    base = s * RPT

    for tbl, outp in ((t0_hbm, outA), (t1_hbm, outB)):
        _zero_2d(bufs[0], K, HW)
        for j in range(RPT // K):
            pltpu.sync_copy(bufs[0], shared.at[pl.ds(base + j * K, K)])
        plsc.subcore_barrier()

        buf0, buf1 = bufs[0], bufs[1]
        sem0, sem1 = gsems[0], gsems[1]
        pltpu.async_copy(tbl.at[sidx.at[0]], buf0, sem0)

        @pl.loop(0, NB // 2 - 1)
        def _(t):
            b = 2 * t
            pltpu.make_async_copy(tbl.at[sidx.at[b]], buf0, sem0).wait()
            pltpu.async_copy(tbl.at[sidx.at[b + 1]], buf1, sem1)
            pltpu.sync_copy(buf0, shared.at[didx.at[b]], add=True)
            pltpu.make_async_copy(tbl.at[sidx.at[b + 1]], buf1, sem1).wait()
            pltpu.async_copy(tbl.at[sidx.at[b + 2]], buf0, sem0)
            pltpu.sync_copy(buf1, shared.at[didx.at[b + 1]], add=True)

        b = NB - 2
        pltpu.make_async_copy(tbl.at[sidx.at[b]], buf0, sem0).wait()
        pltpu.async_copy(tbl.at[sidx.at[b + 1]], buf1, sem1)
        pltpu.sync_copy(buf0, shared.at[didx.at[b]], add=True)
        pltpu.make_async_copy(tbl.at[sidx.at[b + 1]], buf1, sem1).wait()
        pltpu.sync_copy(buf1, shared.at[didx.at[b + 1]], add=True)

        plsc.subcore_barrier()
        pltpu.sync_copy(shared.at[pl.ds(base, RPT)],
                        outp.at[c, pl.ds(base, RPT)])


def _tc_call(body, out_shapes):
    return pl.pallas_call(body, out_shape=out_shapes)


def _gelu(v):
    return 0.5 * v * (1.0 + lax.erf(v * (2.0 ** -0.5)))


def _cat(a0A, a1A, a0B, a1B):
    return jnp.concatenate([a0A[...] + a1A[...], a0B[...] + a1B[...]], axis=1)


def _tc1_body(d0, d1, x, w1, dis_o, hs1_o):
    deg = d0[...] + d1[...] + 1.0
    dis = lax.rsqrt(deg)
    h = jnp.dot(x[...], w1[...], preferred_element_type=_f32)
    dis_o[...] = dis
    hs1_o[...] = h * dis


def _tc2_body(a0A, a1A, a0B, a1B, hs1, dis, b1, w2, x1_o, hs2_o):
    agg = _cat(a0A, a1A, a0B, a1B) + hs1[...]
    x1 = _gelu(dis[...] * agg + b1[...])
    h2 = jnp.dot(x1, w2[...], preferred_element_type=_f32)
    x1_o[...] = x1
    hs2_o[...] = h2 * dis[...]


def _tc3_body(a0A, a1A, a0B, a1B, hs2, dis, b2, x1, lnw, lnb, wp,
              xl_o, hsp_o):
    agg = _cat(a0A, a1A, a0B, a1B) + hs2[...]
    x2 = _gelu(dis[...] * agg + b2[...])
    y = x1[...] + x2
    u = jnp.mean(y, axis=1, keepdims=True)
    v = jnp.mean((y - u) ** 2, axis=1, keepdims=True)
    xl = lnw[...] * ((y - u) / jnp.sqrt(v + 1e-12)) + lnb[...]
    sp = jnp.dot(xl, wp[...], preferred_element_type=_f32)
    xl_o[...] = xl
    hsp_o[...] = sp * dis[...]


def _tc4_body(a0, a1, hsp, dis, bp, batch, xl, xp_o, mask_o):
    spre = dis[...] * (a0[...] + a1[...] + hsp[...]) + bp[0, 0]
    bcol = batch[...]
    gid = lax.broadcasted_iota(jnp.int32, (NP, G), 1)
    onehot = bcol == gid
    valid = bcol < G
    NEG = jnp.float32(-1e30)
    m = jnp.max(jnp.where(onehot, spre, NEG), axis=0, keepdims=True)
    mb = jnp.sum(jnp.where(onehot, m, 0.0), axis=1, keepdims=True)
    ex = jnp.exp(spre - mb)
    den = jnp.sum(jnp.where(onehot, ex, 0.0), axis=0, keepdims=True)
    denb = jnp.sum(jnp.where(onehot, den, 0.0), axis=1, keepdims=True)
    score = jnp.where(valid, ex / (denb + 1e-16), 0.0)
    smax = jnp.max(jnp.where(onehot, score, NEG), axis=0, keepdims=True)
    smaxb = jnp.sum(jnp.where(onehot, smax, 0.0), axis=1, keepdims=True)
    smin = jnp.minimum(smaxb - 1e-7, 0.001)
    mask = (score > smin) & valid
    xp_o[...] = xl[...] * score
    mask_o[...] = mask.astype(_f32)


def _tc5_body(d0, d1, xp, g_o, dis3_o):
    deg3 = d0[...] + d1[...] + 1.0
    dis3 = lax.rsqrt(deg3)
    dis3_o[...] = dis3
    g_o[...] = xp[...] * dis3


def _tc7_body(a0A, a1A, a0B, a1B, g, dis3, w3, b3, mask, batch_row, out_o):
    agg = _cat(a0A, a1A, a0B, a1B) + g[...]
    h3 = jnp.dot(agg, w3[...], preferred_element_type=_f32)
    x3 = _gelu(dis3[...] * h3 + b3[...])
    x3 = jnp.where(mask[...] > 0.5, x3, 0.0)
    gid = lax.broadcasted_iota(jnp.int32, (G, NP), 0)
    onehot_t = (batch_row[...] == gid).astype(_f32)
    out_o[...] = jnp.dot(onehot_t, x3, preferred_element_type=_f32)


def kernel(x, edge_index, batch, W1, b1, W2, b2, ln_w, ln_b, Wp, bp, W3, b3):
    npad = EP - E
    iota_pad = jnp.arange(npad, dtype=jnp.int32)
    trash_ids = TRASH + (iota_pad % (NP - N))
    src = jnp.concatenate([edge_index[0], iota_pad % N])
    dst = jnp.concatenate([edge_index[1], trash_ids])
    src2 = src.reshape(NW, EPT)
    dst2 = dst.reshape(NW, EPT)
    src3 = src.reshape(NW, NB, K)
    dst3 = dst.reshape(NW, NB, K)
    xpad = jnp.pad(x, ((0, NP - N), (0, 0)))
    batch_p = jnp.pad(batch, (0, NP - N), constant_values=G)
    batch_col = batch_p.reshape(NP, 1)
    batch_row = batch_p.reshape(1, NP)
    b1r = b1.reshape(1, F)
    b2r = b2.reshape(1, F)
    lnwr = ln_w.reshape(1, F)
    lnbr = ln_b.reshape(1, F)
    bpr = bp.reshape(1, 1)
    b3r = b3.reshape(1, H4)

    deg = _sc_deg(dst2)
    d0 = deg[0].reshape(NP, 1)
    d1 = deg[1].reshape(NP, 1)

    dis, hs1 = _tc_call(_tc1_body, (
        jax.ShapeDtypeStruct((NP, 1), _f32),
        jax.ShapeDtypeStruct((NP, F), _f32)))(d0, d1, xpad, W1)

    aggA, aggB = _sc_conv(hs1[:, :HW], hs1[:, HW:], src3, dst3)
    x1, hs2 = _tc_call(_tc2_body, (
        jax.ShapeDtypeStruct((NP, F), _f32),
        jax.ShapeDtypeStruct((NP, F), _f32)))(
            aggA[0], aggA[1], aggB[0], aggB[1], hs1, dis, b1r, W2)

    aggA, aggB = _sc_conv(hs2[:, :HW], hs2[:, HW:], src3, dst3)
    xl, hsp = _tc_call(_tc3_body, (
        jax.ShapeDtypeStruct((NP, F), _f32),
        jax.ShapeDtypeStruct((NP, 1), _f32)))(
            aggA[0], aggA[1], aggB[0], aggB[1], hs2, dis, b2r, x1,
            lnwr, lnbr, Wp)

    aggp = _sc_hist_gather(hsp.reshape(NP), src2, dst2)
    xp, maskf = _tc_call(_tc4_body, (
        jax.ShapeDtypeStruct((NP, F), _f32),
        jax.ShapeDtypeStruct((NP, 1), _f32)))(
            aggp[0].reshape(NP, 1), aggp[1].reshape(NP, 1), hsp, dis, bpr,
            batch_col, xl)

    deg3, srcp, dstp = _sc_prune(src2, dst2, maskf.reshape(NP))
    g, dis3 = _tc_call(_tc5_body, (
        jax.ShapeDtypeStruct((NP, F), _f32),
        jax.ShapeDtypeStruct((NP, 1), _f32)))(
            deg3[0].reshape(NP, 1), deg3[1].reshape(NP, 1), xp)

    aggA, aggB = _sc_conv(g[:, :HW], g[:, HW:],
                          srcp.reshape(NW, NB, K), dstp.reshape(NW, NB, K))
    out = _tc_call(_tc7_body, jax.ShapeDtypeStruct((G, H4), _f32))(
        aggA[0], aggA[1], aggB[0], aggB[1], g, dis3, W3, b3r, maskf,
        batch_row)
    return out


# speedup vs baseline: 2.2071x; 1.3322x over previous
import functools

import jax
import jax.numpy as jnp
from jax import lax
from jax.experimental import pallas as pl
from jax.experimental.pallas import tpu as pltpu
from jax.experimental.pallas import tpu_sc as plsc

N = 10000
E = 320000
G = 64
F = 128
H4 = 32
NP = 10240
HW = 64
TRASH = N

NC, NS = 2, 16
NW = NC * NS
EP = NW * NP
EPT = EP // NW
K = 128
NB = EPT // K
RPT = NP // NS
NRING = 2

_f32 = jnp.float32

_MESH = plsc.VectorSubcoreMesh(
    core_axis_name="c", subcore_axis_name="s", num_cores=NC, num_subcores=NS)
_SC_PARAMS = pltpu.CompilerParams(
    needs_layout_passes=False, use_tc_tiling_on_sc=False)


def _zero_1d(ref, nwords):
    z = jnp.zeros((16,), _f32)

    @pl.loop(0, nwords // 16, unroll=8)
    def _(i):
        ref[pl.ds(i * 16, 16)] = z


def _zero_2d(ref, rows, cols):
    z = jnp.zeros((16,), _f32)

    @pl.loop(0, rows)
    def _(r):
        for j in range(cols // 16):
            ref[r, pl.ds(j * 16, 16)] = z


def _hist_reduce_write(acc, shared, slab, out, c, s):
    pltpu.sync_copy(acc, shared.at[s])
    plsc.subcore_barrier()
    base = s * RPT
    pltpu.sync_copy(shared.at[:, pl.ds(base, RPT)], slab)

    @pl.loop(0, RPT // 16)
    def _(i):
        v = jnp.zeros((16,), _f32)
        for k in range(NS):
            v = v + slab[k, pl.ds(i * 16, 16)]
        acc[pl.ds(base + i * 16, 16)] = v

    pltpu.sync_copy(acc.at[pl.ds(base, RPT)], out.at[c, pl.ds(base, RPT)])


@functools.partial(
    pl.kernel,
    out_type=jax.ShapeDtypeStruct((NC, NP), _f32),
    mesh=_MESH,
    compiler_params=_SC_PARAMS,
    scratch_types=[
        pltpu.VMEM((EPT,), jnp.int32),
        pltpu.VMEM((NP,), _f32),
        pltpu.VMEM((NS, RPT), _f32),
        pltpu.VMEM_SHARED((NS, NP), _f32),
    ],
)
def _sc_deg(dst_hbm, out, dstv, acc, slab, shared):
    c = lax.axis_index("c")
    s = lax.axis_index("s")
    wid = c * NS + s
    pltpu.sync_copy(dst_hbm.at[wid], dstv)
    _zero_1d(acc, NP)
    one = jnp.ones((16,), _f32)

    @pl.loop(0, EPT // 16, unroll=4)
    def _(i):
        d = dstv[pl.ds(i * 16, 16)]
        plsc.addupdate_scatter(acc, [d], one)

    _hist_reduce_write(acc, shared, slab, out, c, s)


@functools.partial(
    pl.kernel,
    out_type=jax.ShapeDtypeStruct((NC, NP), _f32),
    mesh=_MESH,
    compiler_params=_SC_PARAMS,
    scratch_types=[
        pltpu.VMEM((EPT,), jnp.int32),
        pltpu.VMEM((EPT,), jnp.int32),
        pltpu.VMEM((NP,), _f32),
        pltpu.VMEM((NP,), _f32),
        pltpu.VMEM((NS, RPT), _f32),
        pltpu.VMEM_SHARED((NS, NP), _f32),
    ],
)
def _sc_hist_gather(val_hbm, src_hbm, dst_hbm, out, srcv, dstv, valv, acc,
                    slab, shared):
    c = lax.axis_index("c")
    s = lax.axis_index("s")
    wid = c * NS + s
    pltpu.sync_copy(src_hbm.at[wid], srcv)
    pltpu.sync_copy(dst_hbm.at[wid], dstv)
    pltpu.sync_copy(val_hbm, valv)
    _zero_1d(acc, NP)

    @pl.loop(0, EPT // 16, unroll=4)
    def _(i):
        sv = srcv[pl.ds(i * 16, 16)]
        d = dstv[pl.ds(i * 16, 16)]
        v = plsc.load_gather(valv, [sv])
        plsc.addupdate_scatter(acc, [d], v)

    _hist_reduce_write(acc, shared, slab, out, c, s)


@functools.partial(
    pl.kernel,
    out_type=(
        jax.ShapeDtypeStruct((NC, NP), _f32),
        jax.ShapeDtypeStruct((NW, EPT), jnp.int32),
    ),
    mesh=_MESH,
    compiler_params=_SC_PARAMS,
    scratch_types=[
        pltpu.VMEM((EPT,), jnp.int32),
        pltpu.VMEM((EPT,), jnp.int32),
        pltpu.VMEM((EPT,), jnp.int32),
        pltpu.VMEM((NP,), _f32),
        pltpu.VMEM((NP,), _f32),
        pltpu.VMEM((NS, RPT), _f32),
        pltpu.VMEM_SHARED((NS, NP), _f32),
    ],
)
def _sc_prune(src_hbm, dst_hbm, mask_hbm, out_deg, out_dst,
              srcv, dstv, dpv, maskv, acc, slab, shared):
    c = lax.axis_index("c")
    s = lax.axis_index("s")
    wid = c * NS + s
    pltpu.sync_copy(src_hbm.at[wid], srcv)
    pltpu.sync_copy(dst_hbm.at[wid], dstv)
    pltpu.sync_copy(mask_hbm, maskv)
    _zero_1d(acc, NP)
    one = jnp.ones((16,), _f32)
    lane = lax.iota(jnp.int32, 16)

    @pl.loop(0, EPT // 16, unroll=4)
    def _(i):
        sv = srcv[pl.ds(i * 16, 16)]
        d = dstv[pl.ds(i * 16, 16)]
        ms = plsc.load_gather(maskv, [sv])
        md = plsc.load_gather(maskv, [d])
        em = (ms > 0.5) & (md > 0.5)
        trash = TRASH + lax.rem(i, 15) * 16 + lane
        dp = jnp.where(em, d, trash)
        dpv[pl.ds(i * 16, 16)] = dp
        plsc.addupdate_scatter(acc, [dp], one)

    pltpu.sync_copy(dpv, out_dst.at[wid])
    _hist_reduce_write(acc, shared, slab, out_deg, c, s)


@functools.partial(
    pl.kernel,
    out_type=(
        jax.ShapeDtypeStruct((NC, NP, HW), _f32),
        jax.ShapeDtypeStruct((NC, NP, HW), _f32),
    ),
    mesh=_MESH,
    compiler_params=_SC_PARAMS,
    scratch_types=[
        pltpu.VMEM((NB, K), jnp.int32),
        pltpu.VMEM((NB, K), jnp.int32),
        [pltpu.VMEM((K, HW), _f32)] * NRING,
        [pltpu.SemaphoreType.DMA] * NRING,
        [pltpu.SemaphoreType.DMA] * NRING,
        pltpu.VMEM_SHARED((NP, HW), _f32),
    ],
)
def _sc_conv(t0_hbm, t1_hbm, src_hbm, dst_hbm, outA, outB, sidx, didx,
             bufs, gsems, ssems, shared):
    c = lax.axis_index("c")
    s = lax.axis_index("s")
    wid = c * NS + s
    pltpu.sync_copy(src_hbm.at[wid], sidx)
    pltpu.sync_copy(dst_hbm.at[wid], didx)
    base = s * RPT

    for tbl, outp in ((t0_hbm, outA), (t1_hbm, outB)):
        _zero_2d(bufs[0], K, HW)
        for j in range(RPT // K):
            pltpu.sync_copy(bufs[0], shared.at[pl.ds(base + j * K, K)])
        plsc.subcore_barrier()

        buf0, buf1 = bufs[0], bufs[1]
        sem0, sem1 = gsems[0], gsems[1]
        pltpu.async_copy(tbl.at[sidx.at[0]], buf0, sem0)

        @pl.loop(0, NB // 2 - 1)
        def _(t):
            b = 2 * t
            pltpu.make_async_copy(tbl.at[sidx.at[b]], buf0, sem0).wait()
            pltpu.async_copy(tbl.at[sidx.at[b + 1]], buf1, sem1)
            pltpu.sync_copy(buf0, shared.at[didx.at[b]], add=True)
            pltpu.make_async_copy(tbl.at[sidx.at[b + 1]], buf1, sem1).wait()
            pltpu.async_copy(tbl.at[sidx.at[b + 2]], buf0, sem0)
            pltpu.sync_copy(buf1, shared.at[didx.at[b + 1]], add=True)

        b = NB - 2
        pltpu.make_async_copy(tbl.at[sidx.at[b]], buf0, sem0).wait()
        pltpu.async_copy(tbl.at[sidx.at[b + 1]], buf1, sem1)
        pltpu.sync_copy(buf0, shared.at[didx.at[b]], add=True)
        pltpu.make_async_copy(tbl.at[sidx.at[b + 1]], buf1, sem1).wait()
        pltpu.sync_copy(buf1, shared.at[didx.at[b + 1]], add=True)

        plsc.subcore_barrier()
        pltpu.sync_copy(shared.at[pl.ds(base, RPT)],
                        outp.at[c, pl.ds(base, RPT)])


def _tc_call(body, out_shapes):
    return pl.pallas_call(body, out_shape=out_shapes)


def _gelu(v):
    return 0.5 * v * (1.0 + lax.erf(v * (2.0 ** -0.5)))


def _cat(a0A, a1A, a0B, a1B):
    return jnp.concatenate([a0A[...] + a1A[...], a0B[...] + a1B[...]], axis=1)


def _tc1_body(d0, d1, x, w1, dis_o, hs1_o):
    deg = d0[...] + d1[...] + 1.0
    dis = lax.rsqrt(deg)
    h = jnp.dot(x[...], w1[...], preferred_element_type=_f32)
    dis_o[...] = dis
    hs1_o[...] = h * dis


def _tc2_body(a0A, a1A, a0B, a1B, hs1, dis, b1, w2, x1_o, hs2_o):
    agg = _cat(a0A, a1A, a0B, a1B) + hs1[...]
    x1 = _gelu(dis[...] * agg + b1[...])
    h2 = jnp.dot(x1, w2[...], preferred_element_type=_f32)
    x1_o[...] = x1
    hs2_o[...] = h2 * dis[...]


def _tc3_body(a0A, a1A, a0B, a1B, hs2, dis, b2, x1, lnw, lnb, wp,
              xl_o, hsp_o):
    agg = _cat(a0A, a1A, a0B, a1B) + hs2[...]
    x2 = _gelu(dis[...] * agg + b2[...])
    y = x1[...] + x2
    u = jnp.mean(y, axis=1, keepdims=True)
    v = jnp.mean((y - u) ** 2, axis=1, keepdims=True)
    xl = lnw[...] * ((y - u) / jnp.sqrt(v + 1e-12)) + lnb[...]
    sp = jnp.dot(xl, wp[...], preferred_element_type=_f32)
    xl_o[...] = xl
    hsp_o[...] = sp * dis[...]


def _tc4_body(a0, a1, hsp, dis, bp, batch, xl, xp_o, mask_o):
    spre = dis[...] * (a0[...] + a1[...] + hsp[...]) + bp[0, 0]
    bcol = batch[...]
    gid = lax.broadcasted_iota(jnp.int32, (NP, G), 1)
    onehot = bcol == gid
    valid = bcol < G
    NEG = jnp.float32(-1e30)
    m = jnp.max(jnp.where(onehot, spre, NEG), axis=0, keepdims=True)
    mb = jnp.sum(jnp.where(onehot, m, 0.0), axis=1, keepdims=True)
    ex = jnp.exp(spre - mb)
    den = jnp.sum(jnp.where(onehot, ex, 0.0), axis=0, keepdims=True)
    denb = jnp.sum(jnp.where(onehot, den, 0.0), axis=1, keepdims=True)
    score = jnp.where(valid, ex / (denb + 1e-16), 0.0)
    smax = jnp.max(jnp.where(onehot, score, NEG), axis=0, keepdims=True)
    smaxb = jnp.sum(jnp.where(onehot, smax, 0.0), axis=1, keepdims=True)
    smin = jnp.minimum(smaxb - 1e-7, 0.001)
    mask = (score > smin) & valid
    xp_o[...] = xl[...] * score
    mask_o[...] = mask.astype(_f32)


def _tc5_body(d0, d1, xp, g_o, dis3_o):
    deg3 = d0[...] + d1[...] + 1.0
    dis3 = lax.rsqrt(deg3)
    dis3_o[...] = dis3
    g_o[...] = xp[...] * dis3


def _tc7_body(a0A, a1A, a0B, a1B, g, dis3, w3, b3, mask, batch_row, out_o):
    agg = _cat(a0A, a1A, a0B, a1B) + g[...]
    h3 = jnp.dot(agg, w3[...], preferred_element_type=_f32)
    x3 = _gelu(dis3[...] * h3 + b3[...])
    x3 = jnp.where(mask[...] > 0.5, x3, 0.0)
    gid = lax.broadcasted_iota(jnp.int32, (G, NP), 0)
    onehot_t = (batch_row[...] == gid).astype(_f32)
    out_o[...] = jnp.dot(onehot_t, x3, preferred_element_type=_f32)


def kernel(x, edge_index, batch, W1, b1, W2, b2, ln_w, ln_b, Wp, bp, W3, b3):
    npad = EP - E
    iota_pad = jnp.arange(npad, dtype=jnp.int32)
    trash_ids = TRASH + (iota_pad % (NP - N))
    src = jnp.concatenate([edge_index[0], iota_pad % N])
    dst = jnp.concatenate([edge_index[1], trash_ids])
    src2 = src.reshape(NW, EPT)
    dst2 = dst.reshape(NW, EPT)
    src3 = src.reshape(NW, NB, K)
    dst3 = dst.reshape(NW, NB, K)
    xpad = jnp.pad(x, ((0, NP - N), (0, 0)))
    batch_p = jnp.pad(batch, (0, NP - N), constant_values=G)
    batch_col = batch_p.reshape(NP, 1)
    batch_row = batch_p.reshape(1, NP)
    b1r = b1.reshape(1, F)
    b2r = b2.reshape(1, F)
    lnwr = ln_w.reshape(1, F)
    lnbr = ln_b.reshape(1, F)
    bpr = bp.reshape(1, 1)
    b3r = b3.reshape(1, H4)

    deg = _sc_deg(dst2)
    d0 = deg[0].reshape(NP, 1)
    d1 = deg[1].reshape(NP, 1)

    dis, hs1 = _tc_call(_tc1_body, (
        jax.ShapeDtypeStruct((NP, 1), _f32),
        jax.ShapeDtypeStruct((NP, F), _f32)))(d0, d1, xpad, W1)

    aggA, aggB = _sc_conv(hs1[:, :HW], hs1[:, HW:], src3, dst3)
    x1, hs2 = _tc_call(_tc2_body, (
        jax.ShapeDtypeStruct((NP, F), _f32),
        jax.ShapeDtypeStruct((NP, F), _f32)))(
            aggA[0], aggA[1], aggB[0], aggB[1], hs1, dis, b1r, W2)

    aggA, aggB = _sc_conv(hs2[:, :HW], hs2[:, HW:], src3, dst3)
    xl, hsp = _tc_call(_tc3_body, (
        jax.ShapeDtypeStruct((NP, F), _f32),
        jax.ShapeDtypeStruct((NP, 1), _f32)))(
            aggA[0], aggA[1], aggB[0], aggB[1], hs2, dis, b2r, x1,
            lnwr, lnbr, Wp)

    aggp = _sc_hist_gather(hsp.reshape(NP), src2, dst2)
    xp, maskf = _tc_call(_tc4_body, (
        jax.ShapeDtypeStruct((NP, F), _f32),
        jax.ShapeDtypeStruct((NP, 1), _f32)))(
            aggp[0].reshape(NP, 1), aggp[1].reshape(NP, 1), hsp, dis, bpr,
            batch_col, xl)

    deg3, dstp = _sc_prune(src2, dst2, maskf.reshape(NP))
    g, dis3 = _tc_call(_tc5_body, (
        jax.ShapeDtypeStruct((NP, F), _f32),
        jax.ShapeDtypeStruct((NP, 1), _f32)))(
            deg3[0].reshape(NP, 1), deg3[1].reshape(NP, 1), xp)

    aggA, aggB = _sc_conv(g[:, :HW], g[:, HW:],
                          src3, dstp.reshape(NW, NB, K))
    out = _tc_call(_tc7_body, jax.ShapeDtypeStruct((G, H4), _f32))(
        aggA[0], aggA[1], aggB[0], aggB[1], g, dis3, W3, b3r, maskf,
        batch_row)
    return out


# speedup vs baseline: 2.2635x; 1.0255x over previous
import functools

import jax
import jax.numpy as jnp
from jax import lax
from jax.experimental import pallas as pl
from jax.experimental.pallas import tpu as pltpu
from jax.experimental.pallas import tpu_sc as plsc

N = 10000
E = 320000
G = 64
F = 128
H4 = 32
NP = 10240
HW = 64
TRASH = N

NC, NS = 2, 16
NW = NC * NS
EP = NW * NP
EPT = EP // NW
K = 128
NB = EPT // K
RPT = NP // NS
NRING = 2

_f32 = jnp.float32

_MESH = plsc.VectorSubcoreMesh(
    core_axis_name="c", subcore_axis_name="s", num_cores=NC, num_subcores=NS)
_SC_PARAMS = pltpu.CompilerParams(
    needs_layout_passes=False, use_tc_tiling_on_sc=False)


def _zero_1d(ref, nwords):
    z = jnp.zeros((16,), _f32)

    @pl.loop(0, nwords // 16, unroll=8)
    def _(i):
        ref[pl.ds(i * 16, 16)] = z


def _zero_2d(ref, rows, cols):
    z = jnp.zeros((16,), _f32)

    @pl.loop(0, rows)
    def _(r):
        for j in range(cols // 16):
            ref[r, pl.ds(j * 16, 16)] = z


def _hist_reduce_write(acc, shared, slab, out, c, s):
    pltpu.sync_copy(acc, shared.at[s])
    plsc.subcore_barrier()
    base = s * RPT
    pltpu.sync_copy(shared.at[:, pl.ds(base, RPT)], slab)

    @pl.loop(0, RPT // 16)
    def _(i):
        v = jnp.zeros((16,), _f32)
        for k in range(NS):
            v = v + slab[k, pl.ds(i * 16, 16)]
        acc[pl.ds(base + i * 16, 16)] = v

    pltpu.sync_copy(acc.at[pl.ds(base, RPT)], out.at[c, pl.ds(base, RPT)])


@functools.partial(
    pl.kernel,
    out_type=jax.ShapeDtypeStruct((NC, NP), _f32),
    mesh=_MESH,
    compiler_params=_SC_PARAMS,
    scratch_types=[
        pltpu.VMEM((EPT,), jnp.int32),
        pltpu.VMEM((NP,), _f32),
        pltpu.VMEM((NS, RPT), _f32),
        pltpu.VMEM_SHARED((NS, NP), _f32),
    ],
)
def _sc_deg(dst_hbm, out, dstv, acc, slab, shared):
    c = lax.axis_index("c")
    s = lax.axis_index("s")
    wid = c * NS + s
    pltpu.sync_copy(dst_hbm.at[wid], dstv)
    _zero_1d(acc, NP)
    one = jnp.ones((16,), _f32)

    @pl.loop(0, EPT // 16, unroll=4)
    def _(i):
        d = dstv[pl.ds(i * 16, 16)]
        plsc.addupdate_scatter(acc, [d], one)

    _hist_reduce_write(acc, shared, slab, out, c, s)


@functools.partial(
    pl.kernel,
    out_type=jax.ShapeDtypeStruct((NC, NP), _f32),
    mesh=_MESH,
    compiler_params=_SC_PARAMS,
    scratch_types=[
        pltpu.VMEM((EPT,), jnp.int32),
        pltpu.VMEM((EPT,), jnp.int32),
        pltpu.VMEM((NP,), _f32),
        pltpu.VMEM((NP,), _f32),
        pltpu.VMEM((NS, RPT), _f32),
        pltpu.VMEM_SHARED((NS, NP), _f32),
    ],
)
def _sc_hist_gather(val_hbm, src_hbm, dst_hbm, out, srcv, dstv, valv, acc,
                    slab, shared):
    c = lax.axis_index("c")
    s = lax.axis_index("s")
    wid = c * NS + s
    pltpu.sync_copy(src_hbm.at[wid], srcv)
    pltpu.sync_copy(dst_hbm.at[wid], dstv)
    pltpu.sync_copy(val_hbm, valv)
    _zero_1d(acc, NP)

    @pl.loop(0, EPT // 16, unroll=4)
    def _(i):
        sv = srcv[pl.ds(i * 16, 16)]
        d = dstv[pl.ds(i * 16, 16)]
        v = plsc.load_gather(valv, [sv])
        plsc.addupdate_scatter(acc, [d], v)

    _hist_reduce_write(acc, shared, slab, out, c, s)


@functools.partial(
    pl.kernel,
    out_type=(
        jax.ShapeDtypeStruct((NC, NP), _f32),
        jax.ShapeDtypeStruct((NW, EPT), jnp.int32),
    ),
    mesh=_MESH,
    compiler_params=_SC_PARAMS,
    scratch_types=[
        pltpu.VMEM((EPT,), jnp.int32),
        pltpu.VMEM((EPT,), jnp.int32),
        pltpu.VMEM((EPT,), jnp.int32),
        pltpu.VMEM((NP,), _f32),
        pltpu.VMEM((NP,), _f32),
        pltpu.VMEM((NS, RPT), _f32),
        pltpu.VMEM_SHARED((NS, NP), _f32),
    ],
)
def _sc_prune(src_hbm, dst_hbm, mask_hbm, out_deg, out_dst,
              srcv, dstv, dpv, maskv, acc, slab, shared):
    c = lax.axis_index("c")
    s = lax.axis_index("s")
    wid = c * NS + s
    pltpu.sync_copy(src_hbm.at[wid], srcv)
    pltpu.sync_copy(dst_hbm.at[wid], dstv)
    pltpu.sync_copy(mask_hbm, maskv)
    _zero_1d(acc, NP)
    one = jnp.ones((16,), _f32)
    lane = lax.iota(jnp.int32, 16)

    @pl.loop(0, EPT // 16, unroll=4)
    def _(i):
        sv = srcv[pl.ds(i * 16, 16)]
        d = dstv[pl.ds(i * 16, 16)]
        ms = plsc.load_gather(maskv, [sv])
        md = plsc.load_gather(maskv, [d])
        em = (ms > 0.5) & (md > 0.5)
        trash = TRASH + lax.rem(i, 15) * 16 + lane
        dp = jnp.where(em, d, trash)
        dpv[pl.ds(i * 16, 16)] = dp
        plsc.addupdate_scatter(acc, [dp], one)

    pltpu.sync_copy(dpv, out_dst.at[wid])
    _hist_reduce_write(acc, shared, slab, out_deg, c, s)


@functools.partial(
    pl.kernel,
    out_type=(
        jax.ShapeDtypeStruct((NC, NP, HW), _f32),
        jax.ShapeDtypeStruct((NC, NP, HW), _f32),
    ),
    mesh=_MESH,
    compiler_params=_SC_PARAMS,
    scratch_types=[
        pltpu.VMEM((NB, K), jnp.int32),
        pltpu.VMEM((NB, K), jnp.int32),
        [pltpu.VMEM((K, HW), _f32)] * NRING,
        [pltpu.SemaphoreType.DMA] * NRING,
        [pltpu.SemaphoreType.DMA] * NRING,
        pltpu.VMEM_SHARED((NP, HW), _f32),
    ],
)
def _sc_conv(t0_hbm, t1_hbm, src_hbm, dst_hbm, outA, outB, sidx, didx,
             bufs, gsems, ssems, shared):
    c = lax.axis_index("c")
    s = lax.axis_index("s")
    wid = c * NS + s
    pltpu.sync_copy(src_hbm.at[wid], sidx)
    pltpu.sync_copy(dst_hbm.at[wid], didx)
    base = s * RPT

    for tbl, outp in ((t0_hbm, outA), (t1_hbm, outB)):
        _zero_2d(bufs[0], K, HW)
        for j in range(RPT // K):
            pltpu.sync_copy(bufs[0], shared.at[pl.ds(base + j * K, K)])
        plsc.subcore_barrier()

        buf0, buf1 = bufs[0], bufs[1]
        g0, g1 = gsems[0], gsems[1]
        s0, s1 = ssems[0], ssems[1]
        pltpu.async_copy(tbl.at[sidx.at[0]], buf0, g0)
        pltpu.async_copy(tbl.at[sidx.at[1]], buf1, g1)

        @pl.loop(0, NB // 2 - 1)
        def _(t):
            b = 2 * t
            pltpu.make_async_copy(tbl.at[sidx.at[b]], buf0, g0).wait()
            pltpu.async_copy(buf0, shared.at[didx.at[b]], s0, add=True)
            pltpu.make_async_copy(tbl.at[sidx.at[b + 1]], buf1, g1).wait()
            pltpu.async_copy(buf1, shared.at[didx.at[b + 1]], s1, add=True)
            pltpu.make_async_copy(buf0, shared.at[didx.at[b]], s0).wait()
            pltpu.async_copy(tbl.at[sidx.at[b + 2]], buf0, g0)
            pltpu.make_async_copy(buf1, shared.at[didx.at[b + 1]], s1).wait()
            pltpu.async_copy(tbl.at[sidx.at[b + 3]], buf1, g1)

        b = NB - 2
        pltpu.make_async_copy(tbl.at[sidx.at[b]], buf0, g0).wait()
        pltpu.async_copy(buf0, shared.at[didx.at[b]], s0, add=True)
        pltpu.make_async_copy(tbl.at[sidx.at[b + 1]], buf1, g1).wait()
        pltpu.async_copy(buf1, shared.at[didx.at[b + 1]], s1, add=True)
        pltpu.make_async_copy(buf0, shared.at[didx.at[b]], s0).wait()
        pltpu.make_async_copy(buf1, shared.at[didx.at[b + 1]], s1).wait()

        plsc.subcore_barrier()
        pltpu.sync_copy(shared.at[pl.ds(base, RPT)],
                        outp.at[c, pl.ds(base, RPT)])


def _tc_call(body, out_shapes):
    return pl.pallas_call(body, out_shape=out_shapes)


def _gelu(v):
    return 0.5 * v * (1.0 + lax.erf(v * (2.0 ** -0.5)))


def _cat(a0A, a1A, a0B, a1B):
    return jnp.concatenate([a0A[...] + a1A[...], a0B[...] + a1B[...]], axis=1)


def _tc1_body(d0, d1, x, w1, dis_o, hs1_o):
    deg = d0[...] + d1[...] + 1.0
    dis = lax.rsqrt(deg)
    h = jnp.dot(x[...], w1[...], preferred_element_type=_f32)
    dis_o[...] = dis
    hs1_o[...] = h * dis


def _tc2_body(a0A, a1A, a0B, a1B, hs1, dis, b1, w2, x1_o, hs2_o):
    agg = _cat(a0A, a1A, a0B, a1B) + hs1[...]
    x1 = _gelu(dis[...] * agg + b1[...])
    h2 = jnp.dot(x1, w2[...], preferred_element_type=_f32)
    x1_o[...] = x1
    hs2_o[...] = h2 * dis[...]


def _tc3_body(a0A, a1A, a0B, a1B, hs2, dis, b2, x1, lnw, lnb, wp,
              xl_o, hsp_o):
    agg = _cat(a0A, a1A, a0B, a1B) + hs2[...]
    x2 = _gelu(dis[...] * agg + b2[...])
    y = x1[...] + x2
    u = jnp.mean(y, axis=1, keepdims=True)
    v = jnp.mean((y - u) ** 2, axis=1, keepdims=True)
    xl = lnw[...] * ((y - u) / jnp.sqrt(v + 1e-12)) + lnb[...]
    sp = jnp.dot(xl, wp[...], preferred_element_type=_f32)
    xl_o[...] = xl
    hsp_o[...] = sp * dis[...]


def _tc4_body(a0, a1, hsp, dis, bp, batch, xl, xp_o, mask_o):
    spre = dis[...] * (a0[...] + a1[...] + hsp[...]) + bp[0, 0]
    bcol = batch[...]
    gid = lax.broadcasted_iota(jnp.int32, (NP, G), 1)
    onehot = bcol == gid
    valid = bcol < G
    NEG = jnp.float32(-1e30)
    m = jnp.max(jnp.where(onehot, spre, NEG), axis=0, keepdims=True)
    mb = jnp.sum(jnp.where(onehot, m, 0.0), axis=1, keepdims=True)
    ex = jnp.exp(spre - mb)
    den = jnp.sum(jnp.where(onehot, ex, 0.0), axis=0, keepdims=True)
    denb = jnp.sum(jnp.where(onehot, den, 0.0), axis=1, keepdims=True)
    score = jnp.where(valid, ex / (denb + 1e-16), 0.0)
    smax = jnp.max(jnp.where(onehot, score, NEG), axis=0, keepdims=True)
    smaxb = jnp.sum(jnp.where(onehot, smax, 0.0), axis=1, keepdims=True)
    smin = jnp.minimum(smaxb - 1e-7, 0.001)
    mask = (score > smin) & valid
    xp_o[...] = xl[...] * score
    mask_o[...] = mask.astype(_f32)


def _tc5_body(d0, d1, xp, g_o, dis3_o):
    deg3 = d0[...] + d1[...] + 1.0
    dis3 = lax.rsqrt(deg3)
    dis3_o[...] = dis3
    g_o[...] = xp[...] * dis3


def _tc7_body(a0A, a1A, a0B, a1B, g, dis3, w3, b3, mask, batch_row, out_o):
    agg = _cat(a0A, a1A, a0B, a1B) + g[...]
    h3 = jnp.dot(agg, w3[...], preferred_element_type=_f32)
    x3 = _gelu(dis3[...] * h3 + b3[...])
    x3 = jnp.where(mask[...] > 0.5, x3, 0.0)
    gid = lax.broadcasted_iota(jnp.int32, (G, NP), 0)
    onehot_t = (batch_row[...] == gid).astype(_f32)
    out_o[...] = jnp.dot(onehot_t, x3, preferred_element_type=_f32)


def kernel(x, edge_index, batch, W1, b1, W2, b2, ln_w, ln_b, Wp, bp, W3, b3):
    npad = EP - E
    iota_pad = jnp.arange(npad, dtype=jnp.int32)
    trash_ids = TRASH + (iota_pad % (NP - N))
    src = jnp.concatenate([edge_index[0], iota_pad % N])
    dst = jnp.concatenate([edge_index[1], trash_ids])
    src2 = src.reshape(NW, EPT)
    dst2 = dst.reshape(NW, EPT)
    src3 = src.reshape(NW, NB, K)
    dst3 = dst.reshape(NW, NB, K)
    xpad = jnp.pad(x, ((0, NP - N), (0, 0)))
    batch_p = jnp.pad(batch, (0, NP - N), constant_values=G)
    batch_col = batch_p.reshape(NP, 1)
    batch_row = batch_p.reshape(1, NP)
    b1r = b1.reshape(1, F)
    b2r = b2.reshape(1, F)
    lnwr = ln_w.reshape(1, F)
    lnbr = ln_b.reshape(1, F)
    bpr = bp.reshape(1, 1)
    b3r = b3.reshape(1, H4)

    deg = _sc_deg(dst2)
    d0 = deg[0].reshape(NP, 1)
    d1 = deg[1].reshape(NP, 1)

    dis, hs1 = _tc_call(_tc1_body, (
        jax.ShapeDtypeStruct((NP, 1), _f32),
        jax.ShapeDtypeStruct((NP, F), _f32)))(d0, d1, xpad, W1)

    aggA, aggB = _sc_conv(hs1[:, :HW], hs1[:, HW:], src3, dst3)
    x1, hs2 = _tc_call(_tc2_body, (
        jax.ShapeDtypeStruct((NP, F), _f32),
        jax.ShapeDtypeStruct((NP, F), _f32)))(
            aggA[0], aggA[1], aggB[0], aggB[1], hs1, dis, b1r, W2)

    aggA, aggB = _sc_conv(hs2[:, :HW], hs2[:, HW:], src3, dst3)
    xl, hsp = _tc_call(_tc3_body, (
        jax.ShapeDtypeStruct((NP, F), _f32),
        jax.ShapeDtypeStruct((NP, 1), _f32)))(
            aggA[0], aggA[1], aggB[0], aggB[1], hs2, dis, b2r, x1,
            lnwr, lnbr, Wp)

    aggp = _sc_hist_gather(hsp.reshape(NP), src2, dst2)
    xp, maskf = _tc_call(_tc4_body, (
        jax.ShapeDtypeStruct((NP, F), _f32),
        jax.ShapeDtypeStruct((NP, 1), _f32)))(
            aggp[0].reshape(NP, 1), aggp[1].reshape(NP, 1), hsp, dis, bpr,
            batch_col, xl)

    deg3, dstp = _sc_prune(src2, dst2, maskf.reshape(NP))
    g, dis3 = _tc_call(_tc5_body, (
        jax.ShapeDtypeStruct((NP, F), _f32),
        jax.ShapeDtypeStruct((NP, 1), _f32)))(
            deg3[0].reshape(NP, 1), deg3[1].reshape(NP, 1), xp)

    aggA, aggB = _sc_conv(g[:, :HW], g[:, HW:],
                          src3, dstp.reshape(NW, NB, K))
    out = _tc_call(_tc7_body, jax.ShapeDtypeStruct((G, H4), _f32))(
        aggA[0], aggA[1], aggB[0], aggB[1], g, dis3, W3, b3r, maskf,
        batch_row)
    return out
